# Initial kernel scaffold; baseline (speedup 1.0000x reference)
import functools

import jax
import jax.numpy as jnp
from jax.experimental import pallas as pl
from jax.experimental.pallas import tpu as pltpu

_B, _H, _W = 16, 384, 384
_C = 32
_HW = _H * _W
_K_TARGET = _HW - int(round(_HW * 0.1))


def _threshold_kernel(vals_ref, out_ref):
    bits = jax.lax.bitcast_convert_type(vals_ref[...], jnp.int32)
    ikey = jnp.where(bits >= 0, bits, bits ^ jnp.int32(0x7FFFFFFF))

    k0 = jnp.full((_B, 1, 1), jnp.iinfo(jnp.int32).min, dtype=jnp.int32)

    def body(j, k):
        trial = k + (jnp.int32(1) << (jnp.int32(31) - j))
        cnt = jnp.sum((ikey < trial).astype(jnp.int32), axis=(1, 2),
                      keepdims=True)
        return jnp.where(cnt < _K_TARGET, trial, k)

    k = jax.lax.fori_loop(0, 32, body, k0)
    dbits = jnp.where(k >= 0, k, k ^ jnp.int32(0x7FFFFFFF))
    d = jax.lax.bitcast_convert_type(dbits, jnp.float32)
    out_ref[...] = jnp.broadcast_to(d.reshape(_B, 1), (_B, 128))


def _mask_kernel(thr_ref, vals_ref, out_ref):
    thr = thr_ref[0, 0]
    out_ref[...] = (vals_ref[...] > thr).astype(jnp.float32)[None]


@jax.jit
def kernel(vals):
    vals3 = vals.reshape(_B, _HW // 128, 128)
    thr = pl.pallas_call(
        _threshold_kernel,
        out_shape=jax.ShapeDtypeStruct((_B, 128), jnp.float32),
    )(vals3)

    out = pl.pallas_call(
        _mask_kernel,
        grid=(_B, _C),
        in_specs=[
            pl.BlockSpec((1, 128), lambda b, c: (b, 0)),
            pl.BlockSpec((1, _H, _W), lambda b, c: (b, 0, 0)),
        ],
        out_specs=pl.BlockSpec((1, 1, _H, _W), lambda b, c: (b, c, 0, 0)),
        out_shape=jax.ShapeDtypeStruct((_B, _C, _H, _W), jnp.float32),
    )(thr, vals)
    return out


# speedup vs baseline: 9.0660x; 9.0660x over previous
import functools

import jax
import jax.numpy as jnp
from jax.experimental import pallas as pl
from jax.experimental.pallas import tpu as pltpu

_B, _H, _W = 16, 384, 384
_C = 32
_HW = _H * _W
_K_TARGET = _HW - int(round(_HW * 0.1))


def _threshold_kernel(vals_ref, out_ref):
    bits = jax.lax.bitcast_convert_type(vals_ref[...], jnp.int32)
    ikey = jnp.where(bits >= 0, bits, bits ^ jnp.int32(0x7FFFFFFF))

    k0 = jnp.full((_B, 1, 1), jnp.iinfo(jnp.int32).min, dtype=jnp.int32)

    def body(j, k):
        trial = k + (jnp.int32(1) << (jnp.int32(31) - j))
        cnt = jnp.sum((ikey < trial).astype(jnp.int32), axis=(1, 2),
                      keepdims=True)
        return jnp.where(cnt < _K_TARGET, trial, k)

    k = jax.lax.fori_loop(0, 32, body, k0)
    dbits = jnp.where(k >= 0, k, k ^ jnp.int32(0x7FFFFFFF))
    d = jax.lax.bitcast_convert_type(dbits, jnp.float32)
    out_ref[...] = jnp.broadcast_to(d.reshape(_B, 1, 1), (_B, 1, 128))


def _mask_kernel(thr_ref, vals_ref, out_ref):
    thr = thr_ref[0, 0, 0]
    out_ref[...] = (vals_ref[...] > thr).astype(jnp.float32)[None]


@jax.jit
def kernel(vals):
    vals3 = vals.reshape(_B, _HW // 128, 128)
    thr = pl.pallas_call(
        _threshold_kernel,
        out_shape=jax.ShapeDtypeStruct((_B, 1, 128), jnp.float32),
    )(vals3)

    out = pl.pallas_call(
        _mask_kernel,
        grid=(_B, _C),
        in_specs=[
            pl.BlockSpec((1, 1, 128), lambda b, c: (b, 0, 0)),
            pl.BlockSpec((1, _H, _W), lambda b, c: (b, 0, 0)),
        ],
        out_specs=pl.BlockSpec((1, 1, _H, _W), lambda b, c: (b, c, 0, 0)),
        out_shape=jax.ShapeDtypeStruct((_B, _C, _H, _W), jnp.float32),
    )(thr, vals)
    return out


# speedup vs baseline: 16.4707x; 1.8168x over previous
import jax
import jax.numpy as jnp
from jax.experimental import pallas as pl
from jax.experimental.pallas import tpu as pltpu

_B, _H, _W = 16, 384, 384
_C = 32
_HW = _H * _W
_ROWS = _HW // 128
_K_TARGET = _HW - int(round(_HW * 0.1))


def _fused_kernel(vals_ref, out_ref, mask_ref, sem):
    b = pl.program_id(0)

    x = vals_ref[...]
    bits = jax.lax.bitcast_convert_type(x, jnp.int32)
    ikey = jnp.where(bits >= 0, bits, bits ^ jnp.int32(0x7FFFFFFF))

    def body(j, k):
        trial = k + (jnp.int32(1) << (jnp.int32(31) - j))
        cnt = jnp.sum((ikey < trial).astype(jnp.int32))
        return jnp.where(cnt < _K_TARGET, trial, k)

    k = jax.lax.fori_loop(0, 32, body, jnp.int32(jnp.iinfo(jnp.int32).min))
    dbits = jnp.where(k >= 0, k, k ^ jnp.int32(0x7FFFFFFF))
    d = jax.lax.bitcast_convert_type(dbits, jnp.float32)

    @pl.when(b > 0)
    def _():
        for c in range(_C):
            pltpu.make_async_copy(mask_ref, out_ref.at[b - 1, c], sem).wait()

    mask_ref[...] = (x.reshape(_H, _W) > d).astype(jnp.float32)
    for c in range(_C):
        pltpu.make_async_copy(mask_ref, out_ref.at[b, c], sem).start()

    @pl.when(b == _B - 1)
    def _():
        for c in range(_C):
            pltpu.make_async_copy(mask_ref, out_ref.at[b, c], sem).wait()


@jax.jit
def kernel(vals):
    vals3 = vals.reshape(_B, _ROWS, 128)
    out = pl.pallas_call(
        _fused_kernel,
        grid=(_B,),
        in_specs=[pl.BlockSpec((1, _ROWS, 128), lambda b: (b, 0, 0))],
        out_specs=pl.BlockSpec(memory_space=pl.ANY),
        out_shape=jax.ShapeDtypeStruct((_B, _C, _H, _W), jnp.float32),
        scratch_shapes=[
            pltpu.VMEM((_H, _W), jnp.float32),
            pltpu.SemaphoreType.DMA,
        ],
    )(vals3)
    return out
